# baseline (speedup 1.0000x reference)
import jax
import jax.numpy as jnp
from jax.experimental import pallas as pl

_BLK = 2048


def _copy_kernel(x_ref, o_ref):
    o_ref[...] = x_ref[...]


def kernel(input):
    b, n, k = input.shape
    rows = b * n * k // 128
    x2 = input.reshape(rows, 128)
    out = pl.pallas_call(
        _copy_kernel,
        grid=(rows // _BLK,),
        in_specs=[pl.BlockSpec((_BLK, 128), lambda i: (i, 0))],
        out_specs=pl.BlockSpec((_BLK, 128), lambda i: (i, 0)),
        out_shape=jax.ShapeDtypeStruct((rows, 128), jnp.float32),
    )(x2)
    return out.reshape(b, n, k)


# speedup vs baseline: 1.3537x; 1.3537x over previous
import jax
import jax.numpy as jnp
from jax.experimental import pallas as pl

_BLK = 4096


def _onehot_argmax_kernel(x_ref, o_ref):
    x = x_ref[...]
    m = jnp.max(x, axis=1, keepdims=True)
    lane = jax.lax.broadcasted_iota(jnp.int32, x.shape, 1)
    idx = jnp.min(jnp.where(x == m, lane, x.shape[1]), axis=1, keepdims=True)
    o_ref[...] = (lane == idx).astype(jnp.float32)


def kernel(input):
    b, n, k = input.shape
    rows = b * n
    x2 = input.reshape(rows, k)
    out = pl.pallas_call(
        _onehot_argmax_kernel,
        grid=(rows // _BLK,),
        in_specs=[pl.BlockSpec((_BLK, k), lambda i: (i, 0))],
        out_specs=pl.BlockSpec((_BLK, k), lambda i: (i, 0)),
        out_shape=jax.ShapeDtypeStruct((rows, k), jnp.float32),
    )(x2)
    return out.reshape(b, n, k)
